# baseline (speedup 1.0000x reference)
import functools

import jax
import jax.numpy as jnp
from jax.experimental import pallas as pl
from jax.experimental.pallas import tpu as pltpu

_BN_EPS = 1e-5
_F32 = jnp.float32
_BF16 = jnp.bfloat16
_HI = jax.lax.Precision.HIGHEST

_PARAMS = pltpu.CompilerParams(
    dimension_semantics=("parallel", "arbitrary"),
    vmem_limit_bytes=48 * 1024 * 1024,
)


def _dot(a, b):
    return jnp.dot(a, b, preferred_element_type=_F32)


def _dot_ta(a, b):
    return jax.lax.dot_general(a, b, (((0,), (0,)), ((), ())),
                               preferred_element_type=_F32)


def _pass1_body(x_ref, w1_ref, b1_ref, s_ref, q_ref, *, bb):
    s = jnp.zeros((1, 128), _F32)
    q = jnp.zeros((1, 128), _F32)
    for i in range(bb):
        pre = _dot_ta(x_ref[i], w1_ref[...]) + b1_ref[...]
        s += jnp.sum(pre, axis=0, keepdims=True)
        q += jnp.sum(pre * pre, axis=0, keepdims=True)
    s_ref[0] = s
    q_ref[0] = q


def _pass2_body(x_ref, w1_ref, a1_ref, w2_ref, b2_ref,
                f_ref, sh_ref, q_ref, mx_ref, mn_ref, *, bb):
    sh = jnp.zeros((1, 128), _F32)
    q = jnp.zeros((1, 256), _F32)
    for i in range(bb):
        h1 = jnp.maximum(_dot_ta(x_ref[i], w1_ref[...]) + a1_ref[...], 0.0)
        sh += jnp.sum(h1, axis=0, keepdims=True)
        pre = _dot(h1, w2_ref[...]) + b2_ref[...]
        f_ref[i] = pre.astype(_BF16)
        q += jnp.sum(pre * pre, axis=0, keepdims=True)
        mx_ref[i] = jnp.max(pre, axis=0, keepdims=True)
        mn_ref[i] = jnp.min(pre, axis=0, keepdims=True)
    sh_ref[0] = sh
    q_ref[0] = q


def _pass3_body(f_ref, w3_ref, gc_ref, q_ref, *, bb):
    q = jnp.zeros((1, 512), _F32)
    for i in range(bb):
        pre = _dot(f_ref[i], w3_ref[...]) + gc_ref[i]
        q += jnp.sum(pre * pre, axis=0, keepdims=True)
    q_ref[0] = q


def _pass4_body(f_ref, w3_ref, gc3_ref, w4_ref, b4_ref,
                sh_ref, q_ref, mx_ref, mn_ref):
    h3 = jnp.maximum(_dot(f_ref[0], w3_ref[...]) + gc3_ref[0], 0.0)
    sh_ref[0] = jnp.sum(h3, axis=0, keepdims=True)
    pre = _dot(h3.astype(_BF16), w4_ref[...]) + b4_ref[...]
    q_ref[0] = jnp.sum(pre * pre, axis=0, keepdims=True)
    mx_ref[0] = jnp.max(pre, axis=0, keepdims=True)
    mn_ref[0] = jnp.min(pre, axis=0, keepdims=True)


def _grid2(g):
    nc = 2 if g % 2 == 0 else 1
    return (nc, g // nc), g // nc


def _row_spec(bb, n, c, h):
    return pl.BlockSpec((bb, n, c), lambda ci, j: (ci * h + j, 0, 0))


def _tile_spec(tn, c, nt, h):
    return pl.BlockSpec((1, tn, c),
                        lambda ci, j: ((ci * h + j) // nt, (ci * h + j) % nt, 0))


def _b_of_tile_spec(c, nt, h):
    return pl.BlockSpec((1, 1, c), lambda ci, j: ((ci * h + j) // nt, 0, 0))


def _per_b_spec(bb, c, h):
    return pl.BlockSpec((bb, 1, c), lambda ci, j: (ci * h + j, 0, 0))


def _step_spec(c, h):
    return pl.BlockSpec((1, 1, c), lambda ci, j: (ci * h + j, 0, 0))


def _full_spec(shape):
    return pl.BlockSpec(shape, lambda ci, j: (0,) * len(shape))


def _stat_shape(steps, c):
    return jax.ShapeDtypeStruct((steps, 1, c), _F32)


def _bn_fold(s, q, count, gamma, beta):
    mean = s / count
    var = jnp.maximum(q / count - mean * mean, 0.0)
    scale = gamma * jax.lax.rsqrt(var + _BN_EPS)
    return scale, beta - mean * scale


def _affine_max(scale, shift, mx, mn):
    return jnp.where(scale > 0, scale * mx + shift, scale * mn + shift)


@jax.jit
def _encode(x_ncw, p):
    B, c_in, N = x_ncw.shape
    fd = p["w4"].shape[1]
    count = jnp.float32(B * N)

    x = x_ncw
    w1 = p["w1"]
    b1, w2, b2, b3, w4, b4 = p["b1"], p["w2"], p["b2"], p["b3"], p["w4"], p["b4"]
    w3g, w3f = p["w3"][:256], p["w3"][256:]

    bb1 = min(16, B)
    g1 = B // bb1
    grid1, h1 = _grid2(g1)
    s1, q1 = pl.pallas_call(
        functools.partial(_pass1_body, bb=bb1),
        grid=grid1,
        in_specs=[_row_spec(bb1, c_in, N, h1), _full_spec((c_in, 128)),
                  _full_spec((1, 128))],
        out_specs=[_step_spec(128, h1), _step_spec(128, h1)],
        out_shape=(_stat_shape(g1, 128), _stat_shape(g1, 128)),
        compiler_params=_PARAMS,
    )(x, w1, b1)
    sc1, sf1 = _bn_fold(jnp.sum(s1, 0), jnp.sum(q1, 0), count,
                        p["g1"], p["be1"])
    w1f = w1 * sc1
    a1 = sc1 * b1 + sf1

    bb2 = min(4, B)
    g2 = B // bb2
    grid2, h2 = _grid2(g2)
    feat, sh1, q2, fmx, fmn = pl.pallas_call(
        functools.partial(_pass2_body, bb=bb2),
        grid=grid2,
        in_specs=[_row_spec(bb2, c_in, N, h2), _full_spec((c_in, 128)),
                  _full_spec((1, 128)), _full_spec((128, 256)),
                  _full_spec((1, 256))],
        out_specs=[_row_spec(bb2, N, 256, h2), _step_spec(128, h2),
                   _step_spec(256, h2),
                   _per_b_spec(bb2, 256, h2), _per_b_spec(bb2, 256, h2)],
        out_shape=(jax.ShapeDtypeStruct((B, N, 256), _BF16),
                   _stat_shape(g2, 128), _stat_shape(g2, 256),
                   jax.ShapeDtypeStruct((B, 1, 256), _F32),
                   jax.ShapeDtypeStruct((B, 1, 256), _F32)),
        compiler_params=_PARAMS,
    )(x, w1f, a1, w2, b2)
    s2 = jnp.dot(jnp.sum(sh1, 0), w2, precision=_HI) + count * b2
    sc2, sf2 = _bn_fold(s2, jnp.sum(q2, 0), count, p["g2"], p["be2"])

    g = _affine_max(sc2, sf2, fmx[:, 0, :], fmn[:, 0, :])
    w3s = sc2.reshape(256, 1) * w3f
    gc = (jnp.dot(g, w3g, precision=_HI)
          + jnp.dot(sf2, w3f, precision=_HI) + b3)
    gc = gc.reshape(B, 1, 512)

    bb3 = min(4, B)
    g3 = B // bb3
    grid3, h3 = _grid2(g3)
    (q3,) = pl.pallas_call(
        functools.partial(_pass3_body, bb=bb3),
        grid=grid3,
        in_specs=[_row_spec(bb3, N, 256, h3), _full_spec((256, 512)),
                  _per_b_spec(bb3, 512, h3)],
        out_specs=[_step_spec(512, h3)],
        out_shape=(_stat_shape(g3, 512),),
        compiler_params=_PARAMS,
    )(feat, w3s.astype(_BF16), gc)
    s3 = (jnp.dot(s2, w3s, precision=_HI)
          + N * jnp.sum(gc[:, 0, :], 0, keepdims=True))
    sc3, sf3 = _bn_fold(s3, jnp.sum(q3, 0), count, p["g3"], p["be3"])
    w34 = (w3s * sc3).astype(_BF16)
    gc3 = gc * sc3.reshape(1, 1, 512) + sf3.reshape(1, 1, 512)

    tn4 = min(2048, N)
    nt4 = N // tn4
    g4 = B * nt4
    grid4, h4 = _grid2(g4)
    sh3, q4, hmx, hmn = pl.pallas_call(
        _pass4_body,
        grid=grid4,
        in_specs=[_tile_spec(tn4, 256, nt4, h4), _full_spec((256, 512)),
                  _b_of_tile_spec(512, nt4, h4), _full_spec((512, fd)),
                  _full_spec((1, fd))],
        out_specs=[_step_spec(512, h4), _step_spec(fd, h4),
                   _step_spec(fd, h4), _step_spec(fd, h4)],
        out_shape=(_stat_shape(g4, 512), _stat_shape(g4, fd),
                   _stat_shape(g4, fd), _stat_shape(g4, fd)),
        compiler_params=_PARAMS,
    )(feat, w34, gc3, w4.astype(_BF16), b4)
    s4 = jnp.dot(jnp.sum(sh3, 0), w4, precision=_HI) + count * b4
    sc4, sf4 = _bn_fold(s4, jnp.sum(q4, 0), count, p["g4"], p["be4"])

    hmx = jnp.max(hmx.reshape(B, nt4, fd), axis=1)
    hmn = jnp.min(hmn.reshape(B, nt4, fd), axis=1)
    return _affine_max(sc4, sf4, hmx, hmn)


def kernel(x, w1, b1, g1, be1, w2, b2, g2, be2,
           w3, b3, g3, be3, w4, b4, g4, be4):
    p = {
        "w1": w1, "b1": b1, "g1": g1, "be1": be1,
        "w2": w2, "b2": b2, "g2": g2, "be2": be2,
        "w3": w3, "b3": b3, "g3": g3, "be3": be3,
        "w4": w4, "b4": b4, "g4": g4, "be4": be4,
    }
    return _encode(x, p)


# speedup vs baseline: 1.0469x; 1.0469x over previous
import functools

import jax
import jax.numpy as jnp
from jax.experimental import pallas as pl
from jax.experimental.pallas import tpu as pltpu

_BN_EPS = 1e-5
_F32 = jnp.float32
_BF16 = jnp.bfloat16
_HI = jax.lax.Precision.HIGHEST

_PARAMS = pltpu.CompilerParams(
    dimension_semantics=("parallel", "arbitrary"),
    vmem_limit_bytes=64 * 1024 * 1024,
)


def _dot(a, b):
    return jnp.dot(a, b, preferred_element_type=_F32)


def _dot_ta(a, b):
    return jax.lax.dot_general(a, b, (((0,), (0,)), ((), ())),
                               preferred_element_type=_F32)


def _pass1_body(x_ref, w1_ref, b1_ref, s_ref, q_ref, *, bb):
    s = jnp.zeros((1, 128), _F32)
    q = jnp.zeros((1, 128), _F32)
    for i in range(bb):
        pre = _dot_ta(x_ref[i], w1_ref[...]) + b1_ref[...]
        s += jnp.sum(pre, axis=0, keepdims=True)
        q += jnp.sum(pre * pre, axis=0, keepdims=True)
    s_ref[0] = s
    q_ref[0] = q


def _pass2_body(x_ref, w1_ref, a1_ref, w2_ref, b2_ref,
                f_ref, sh_ref, q_ref, mx_ref, mn_ref, *, bb):
    sh = jnp.zeros((1, 128), _F32)
    q = jnp.zeros((1, 256), _F32)
    for i in range(bb):
        h1 = jnp.maximum(_dot_ta(x_ref[i], w1_ref[...]) + a1_ref[...], 0.0)
        sh += jnp.sum(h1, axis=0, keepdims=True)
        pre = _dot(h1, w2_ref[...]) + b2_ref[...]
        f_ref[i] = pre.astype(_BF16)
        q += jnp.sum(pre * pre, axis=0, keepdims=True)
        mx_ref[i] = jnp.max(pre, axis=0, keepdims=True)
        mn_ref[i] = jnp.min(pre, axis=0, keepdims=True)
    sh_ref[0] = sh
    q_ref[0] = q


def _pass3_body(f_ref, w3_ref, gc_ref, q_ref, *, bb):
    q = jnp.zeros((1, 512), _F32)
    for i in range(bb):
        pre = _dot(f_ref[i], w3_ref[...]) + gc_ref[i]
        q += jnp.sum(pre * pre, axis=0, keepdims=True)
    q_ref[0] = q


def _pass4_body(f_ref, w3_ref, gc3_ref, w4_ref,
                sh_ref, q_ref, mx_ref, mn_ref):
    h3 = jnp.maximum(_dot(f_ref[0], w3_ref[...]) + gc3_ref[0], 0.0)
    sh_ref[0] = jnp.sum(h3, axis=0, keepdims=True)
    y = _dot(h3.astype(_BF16), w4_ref[...])
    q_ref[0] = jnp.sum(y * y, axis=0, keepdims=True)
    mx_ref[0] = jnp.max(y, axis=0, keepdims=True)
    mn_ref[0] = jnp.min(y, axis=0, keepdims=True)


def _grid2(g):
    nc = 2 if g % 2 == 0 else 1
    return (nc, g // nc), g // nc


def _row_spec(bb, n, c, h):
    return pl.BlockSpec((bb, n, c), lambda ci, j: (ci * h + j, 0, 0))


def _tile_spec(tn, c, nt, h):
    return pl.BlockSpec((1, tn, c),
                        lambda ci, j: ((ci * h + j) // nt, (ci * h + j) % nt, 0))


def _b_of_tile_spec(c, nt, h):
    return pl.BlockSpec((1, 1, c), lambda ci, j: ((ci * h + j) // nt, 0, 0))


def _per_b_spec(bb, c, h):
    return pl.BlockSpec((bb, 1, c), lambda ci, j: (ci * h + j, 0, 0))


def _step_spec(c, h):
    return pl.BlockSpec((1, 1, c), lambda ci, j: (ci * h + j, 0, 0))


def _full_spec(shape):
    return pl.BlockSpec(shape, lambda ci, j: (0,) * len(shape))


def _stat_shape(steps, c):
    return jax.ShapeDtypeStruct((steps, 1, c), _F32)


def _bn_fold(s, q, count, gamma, beta):
    mean = s / count
    var = jnp.maximum(q / count - mean * mean, 0.0)
    scale = gamma * jax.lax.rsqrt(var + _BN_EPS)
    return scale, beta - mean * scale


def _affine_max(scale, shift, mx, mn):
    return jnp.where(scale > 0, scale * mx + shift, scale * mn + shift)


@jax.jit
def _encode(x_ncw, p):
    B, c_in, N = x_ncw.shape
    fd = p["w4"].shape[1]
    count = jnp.float32(B * N)

    x = x_ncw
    w1 = p["w1"]
    b1, w2, b2, b3, w4, b4 = p["b1"], p["w2"], p["b2"], p["b3"], p["w4"], p["b4"]
    w3g, w3f = p["w3"][:256], p["w3"][256:]

    bb1 = min(16, B)
    g1 = B // bb1
    grid1, h1 = _grid2(g1)
    s1, q1 = pl.pallas_call(
        functools.partial(_pass1_body, bb=bb1),
        grid=grid1,
        in_specs=[_row_spec(bb1, c_in, N, h1), _full_spec((c_in, 128)),
                  _full_spec((1, 128))],
        out_specs=[_step_spec(128, h1), _step_spec(128, h1)],
        out_shape=(_stat_shape(g1, 128), _stat_shape(g1, 128)),
        compiler_params=_PARAMS,
    )(x, w1, b1)
    sc1, sf1 = _bn_fold(jnp.sum(s1, 0), jnp.sum(q1, 0), count,
                        p["g1"], p["be1"])
    w1f = w1 * sc1
    a1 = sc1 * b1 + sf1

    bb2 = min(4, B)
    g2 = B // bb2
    grid2, h2 = _grid2(g2)
    feat, sh1, q2, fmx, fmn = pl.pallas_call(
        functools.partial(_pass2_body, bb=bb2),
        grid=grid2,
        in_specs=[_row_spec(bb2, c_in, N, h2), _full_spec((c_in, 128)),
                  _full_spec((1, 128)), _full_spec((128, 256)),
                  _full_spec((1, 256))],
        out_specs=[_row_spec(bb2, N, 256, h2), _step_spec(128, h2),
                   _step_spec(256, h2),
                   _per_b_spec(bb2, 256, h2), _per_b_spec(bb2, 256, h2)],
        out_shape=(jax.ShapeDtypeStruct((B, N, 256), _BF16),
                   _stat_shape(g2, 128), _stat_shape(g2, 256),
                   jax.ShapeDtypeStruct((B, 1, 256), _F32),
                   jax.ShapeDtypeStruct((B, 1, 256), _F32)),
        compiler_params=_PARAMS,
    )(x, w1f, a1, w2, b2)
    s2 = jnp.dot(jnp.sum(sh1, 0), w2, precision=_HI) + count * b2
    sc2, sf2 = _bn_fold(s2, jnp.sum(q2, 0), count, p["g2"], p["be2"])

    g = _affine_max(sc2, sf2, fmx[:, 0, :], fmn[:, 0, :])
    w3s = sc2.reshape(256, 1) * w3f
    gc = (jnp.dot(g, w3g, precision=_HI)
          + jnp.dot(sf2, w3f, precision=_HI) + b3)
    gc = gc.reshape(B, 1, 512)

    bb3 = min(4, B)
    g3 = B // bb3
    grid3, h3 = _grid2(g3)
    (q3,) = pl.pallas_call(
        functools.partial(_pass3_body, bb=bb3),
        grid=grid3,
        in_specs=[_row_spec(bb3, N, 256, h3), _full_spec((256, 512)),
                  _per_b_spec(bb3, 512, h3)],
        out_specs=[_step_spec(512, h3)],
        out_shape=(_stat_shape(g3, 512),),
        compiler_params=_PARAMS,
    )(feat, w3s.astype(_BF16), gc)
    s3 = (jnp.dot(s2, w3s, precision=_HI)
          + N * jnp.sum(gc[:, 0, :], 0, keepdims=True))
    sc3, sf3 = _bn_fold(s3, jnp.sum(q3, 0), count, p["g3"], p["be3"])
    w34 = (w3s * sc3).astype(_BF16)
    gc3 = gc * sc3.reshape(1, 1, 512) + sf3.reshape(1, 1, 512)

    tn4 = min(2048, N)
    nt4 = N // tn4
    g4 = B * nt4
    grid4, h4 = _grid2(g4)
    sh3, q4, hmx, hmn = pl.pallas_call(
        _pass4_body,
        grid=grid4,
        in_specs=[_tile_spec(tn4, 256, nt4, h4), _full_spec((256, 512)),
                  _b_of_tile_spec(512, nt4, h4), _full_spec((512, fd))],
        out_specs=[_step_spec(512, h4), _step_spec(fd, h4),
                   _step_spec(fd, h4), _step_spec(fd, h4)],
        out_shape=(_stat_shape(g4, 512), _stat_shape(g4, fd),
                   _stat_shape(g4, fd), _stat_shape(g4, fd)),
        compiler_params=_PARAMS,
    )(feat, w34, gc3, w4.astype(_BF16))
    sy = jnp.dot(jnp.sum(sh3, 0), w4, precision=_HI)
    s4 = sy + count * b4
    qy = jnp.sum(q4, 0) + 2.0 * b4 * sy + count * b4 * b4
    sc4, sf4 = _bn_fold(s4, qy, count, p["g4"], p["be4"])

    hmx = jnp.max(hmx.reshape(B, nt4, fd), axis=1) + b4
    hmn = jnp.min(hmn.reshape(B, nt4, fd), axis=1) + b4
    return _affine_max(sc4, sf4, hmx, hmn)


def kernel(x, w1, b1, g1, be1, w2, b2, g2, be2,
           w3, b3, g3, be3, w4, b4, g4, be4):
    p = {
        "w1": w1, "b1": b1, "g1": g1, "be1": be1,
        "w2": w2, "b2": b2, "g2": g2, "be2": be2,
        "w3": w3, "b3": b3, "g3": g3, "be3": be3,
        "w4": w4, "b4": b4, "g4": g4, "be4": be4,
    }
    return _encode(x, p)


# speedup vs baseline: 1.0577x; 1.0103x over previous
import functools

import jax
import jax.numpy as jnp
from jax.experimental import pallas as pl
from jax.experimental.pallas import tpu as pltpu

_BN_EPS = 1e-5
_F32 = jnp.float32
_BF16 = jnp.bfloat16
_HI = jax.lax.Precision.HIGHEST

_PARAMS = pltpu.CompilerParams(
    dimension_semantics=("parallel", "arbitrary"),
    vmem_limit_bytes=64 * 1024 * 1024,
)


def _dot(a, b):
    return jnp.dot(a, b, preferred_element_type=_F32)


def _dot_ta(a, b):
    return jax.lax.dot_general(a, b, (((0,), (0,)), ((), ())),
                               preferred_element_type=_F32)


def _pass1_body(x_ref, w1_ref, b1_ref, s_ref, q_ref, *, bb):
    s = jnp.zeros((1, 128), _F32)
    q = jnp.zeros((1, 128), _F32)
    for i in range(bb):
        pre = _dot_ta(x_ref[i], w1_ref[...]) + b1_ref[...]
        s += jnp.sum(pre, axis=0, keepdims=True)
        q += jnp.sum(pre * pre, axis=0, keepdims=True)
    s_ref[0] = s
    q_ref[0] = q


def _pass2_body(x_ref, w1_ref, a1_ref, w2_ref, f_ref, o_ref, *, bb):
    sh = jnp.zeros((1, 128), _F32)
    q = jnp.zeros((1, 256), _F32)
    rows = []
    for i in range(bb):
        h1 = jnp.maximum(_dot_ta(x_ref[i], w1_ref[...]) + a1_ref[...], 0.0)
        sh += jnp.sum(h1, axis=0, keepdims=True)
        y = _dot(h1, w2_ref[...])
        f_ref[i] = y.astype(_BF16)
        q += jnp.sum(y * y, axis=0, keepdims=True)
        rows.append(jnp.max(y, axis=0, keepdims=True))
        rows.append(jnp.min(y, axis=0, keepdims=True))
    rows.append(q)
    rows.append(jnp.pad(sh, ((0, 0), (0, 128))))
    o_ref[0] = jnp.concatenate(rows, axis=0)


def _pass3_body(f_ref, w3_ref, gc_ref, q_ref, *, bb):
    q = jnp.zeros((1, 512), _F32)
    for i in range(bb):
        pre = _dot(f_ref[i], w3_ref[...]) + gc_ref[i]
        q += jnp.sum(pre * pre, axis=0, keepdims=True)
    q_ref[0] = q


def _pass4_body(f_ref, w3_ref, gc3_ref, w4_ref, o_ref, *, fd):
    h3 = jnp.maximum(_dot(f_ref[0], w3_ref[...]) + gc3_ref[0], 0.0)
    sh = jnp.sum(h3, axis=0, keepdims=True)
    y = _dot(h3.astype(_BF16), w4_ref[...])
    o_ref[0] = jnp.concatenate(
        [jnp.sum(y * y, axis=0, keepdims=True),
         jnp.max(y, axis=0, keepdims=True),
         jnp.min(y, axis=0, keepdims=True),
         jnp.pad(sh, ((0, 0), (0, fd - 512)))], axis=0)


def _grid2(g):
    nc = 2 if g % 2 == 0 else 1
    return (nc, g // nc), g // nc


def _row_spec(bb, n, c, h):
    return pl.BlockSpec((bb, n, c), lambda ci, j: (ci * h + j, 0, 0))


def _tile_spec(tn, c, nt, h):
    return pl.BlockSpec((1, tn, c),
                        lambda ci, j: ((ci * h + j) // nt, (ci * h + j) % nt, 0))


def _b_of_tile_spec(c, nt, h):
    return pl.BlockSpec((1, 1, c), lambda ci, j: ((ci * h + j) // nt, 0, 0))


def _per_b_spec(bb, c, h):
    return pl.BlockSpec((bb, 1, c), lambda ci, j: (ci * h + j, 0, 0))


def _step_spec(c, h):
    return pl.BlockSpec((1, 1, c), lambda ci, j: (ci * h + j, 0, 0))


def _full_spec(shape):
    return pl.BlockSpec(shape, lambda ci, j: (0,) * len(shape))


def _stat_shape(steps, c):
    return jax.ShapeDtypeStruct((steps, 1, c), _F32)


def _bn_fold(s, q, count, gamma, beta):
    mean = s / count
    var = jnp.maximum(q / count - mean * mean, 0.0)
    scale = gamma * jax.lax.rsqrt(var + _BN_EPS)
    return scale, beta - mean * scale


def _affine_max(scale, shift, mx, mn):
    return jnp.where(scale > 0, scale * mx + shift, scale * mn + shift)


@jax.jit
def _encode(x_ncw, p):
    B, c_in, N = x_ncw.shape
    fd = p["w4"].shape[1]
    count = jnp.float32(B * N)

    x = x_ncw
    w1 = p["w1"]
    b1, w2, b2, b3, w4, b4 = p["b1"], p["w2"], p["b2"], p["b3"], p["w4"], p["b4"]
    w3g, w3f = p["w3"][:256], p["w3"][256:]

    bb1 = min(16, B)
    g1 = B // bb1
    grid1, h1 = _grid2(g1)
    s1, q1 = pl.pallas_call(
        functools.partial(_pass1_body, bb=bb1),
        grid=grid1,
        in_specs=[_row_spec(bb1, c_in, N, h1), _full_spec((c_in, 128)),
                  _full_spec((1, 128))],
        out_specs=[_step_spec(128, h1), _step_spec(128, h1)],
        out_shape=(_stat_shape(g1, 128), _stat_shape(g1, 128)),
        compiler_params=_PARAMS,
    )(x, w1, b1)
    sc1, sf1 = _bn_fold(jnp.sum(s1, 0), jnp.sum(q1, 0), count,
                        p["g1"], p["be1"])
    w1f = w1 * sc1
    a1 = sc1 * b1 + sf1

    bb2 = min(4, B)
    g2 = B // bb2
    grid2, h2 = _grid2(g2)
    nrow2 = 2 * bb2 + 2
    feat, o2 = pl.pallas_call(
        functools.partial(_pass2_body, bb=bb2),
        grid=grid2,
        in_specs=[_row_spec(bb2, c_in, N, h2), _full_spec((c_in, 128)),
                  _full_spec((1, 128)), _full_spec((128, 256))],
        out_specs=[_row_spec(bb2, N, 256, h2),
                   pl.BlockSpec((1, nrow2, 256),
                                lambda ci, j, _h=h2: (ci * _h + j, 0, 0))],
        out_shape=(jax.ShapeDtypeStruct((B, N, 256), _BF16),
                   jax.ShapeDtypeStruct((g2, nrow2, 256), _F32)),
        compiler_params=_PARAMS,
    )(x, w1f, a1, w2)
    mxmn = o2[:, :2 * bb2, :].reshape(B, 2, 256)
    fmx = mxmn[:, 0, :] + b2
    fmn = mxmn[:, 1, :] + b2
    sh1 = jnp.sum(o2[:, 2 * bb2 + 1, :128], axis=0, keepdims=True)
    sy2 = jnp.dot(sh1, w2, precision=_HI)
    s2 = sy2 + count * b2
    q2 = jnp.sum(o2[:, 2 * bb2, :], 0) + 2.0 * b2 * sy2 + count * b2 * b2
    sc2, sf2 = _bn_fold(s2, q2, count, p["g2"], p["be2"])

    g = _affine_max(sc2, sf2, fmx, fmn)
    w3s = sc2.reshape(256, 1) * w3f
    gc = (jnp.dot(g, w3g, precision=_HI)
          + jnp.dot(sf2, w3f, precision=_HI) + b3
          + jnp.dot(b2, w3s, precision=_HI))
    gc = gc.reshape(B, 1, 512)

    bb3 = min(4, B)
    g3 = B // bb3
    grid3, h3 = _grid2(g3)
    (q3,) = pl.pallas_call(
        functools.partial(_pass3_body, bb=bb3),
        grid=grid3,
        in_specs=[_row_spec(bb3, N, 256, h3), _full_spec((256, 512)),
                  _per_b_spec(bb3, 512, h3)],
        out_specs=[_step_spec(512, h3)],
        out_shape=(_stat_shape(g3, 512),),
        compiler_params=_PARAMS,
    )(feat, w3s.astype(_BF16), gc)
    s3 = (jnp.dot(sy2.reshape(1, 256), w3s, precision=_HI)
          + N * jnp.sum(gc[:, 0, :], 0, keepdims=True))
    sc3, sf3 = _bn_fold(s3, jnp.sum(q3, 0), count, p["g3"], p["be3"])
    w34 = (w3s * sc3).astype(_BF16)
    gc3 = gc * sc3.reshape(1, 1, 512) + sf3.reshape(1, 1, 512)

    tn4 = min(2048, N)
    nt4 = N // tn4
    g4 = B * nt4
    grid4, h4 = _grid2(g4)
    (o4,) = pl.pallas_call(
        functools.partial(_pass4_body, fd=fd),
        grid=grid4,
        in_specs=[_tile_spec(tn4, 256, nt4, h4), _full_spec((256, 512)),
                  _b_of_tile_spec(512, nt4, h4), _full_spec((512, fd))],
        out_specs=[pl.BlockSpec((1, 4, fd),
                                lambda ci, j, _h=h4: (ci * _h + j, 0, 0))],
        out_shape=(jax.ShapeDtypeStruct((g4, 4, fd), _F32),),
        compiler_params=_PARAMS,
    )(feat, w34, gc3, w4.astype(_BF16))
    sh3 = jnp.sum(o4[:, 3, :512], axis=0, keepdims=True)
    sy = jnp.dot(sh3, w4, precision=_HI)
    s4 = sy + count * b4
    qy = jnp.sum(o4[:, 0, :], 0) + 2.0 * b4 * sy + count * b4 * b4
    sc4, sf4 = _bn_fold(s4, qy, count, p["g4"], p["be4"])

    hmx = jnp.max(o4[:, 1, :].reshape(B, nt4, fd), axis=1) + b4
    hmn = jnp.min(o4[:, 2, :].reshape(B, nt4, fd), axis=1) + b4
    return _affine_max(sc4, sf4, hmx, hmn)


def kernel(x, w1, b1, g1, be1, w2, b2, g2, be2,
           w3, b3, g3, be3, w4, b4, g4, be4):
    p = {
        "w1": w1, "b1": b1, "g1": g1, "be1": be1,
        "w2": w2, "b2": b2, "g2": g2, "be2": be2,
        "w3": w3, "b3": b3, "g3": g3, "be3": be3,
        "w4": w4, "b4": b4, "g4": g4, "be4": be4,
    }
    return _encode(x, p)


# speedup vs baseline: 1.0731x; 1.0146x over previous
import functools

import jax
import jax.numpy as jnp
from jax.experimental import pallas as pl
from jax.experimental.pallas import tpu as pltpu

_BN_EPS = 1e-5
_F32 = jnp.float32
_BF16 = jnp.bfloat16
_HI = jax.lax.Precision.HIGHEST

_PARAMS = pltpu.CompilerParams(
    dimension_semantics=("parallel", "arbitrary"),
    vmem_limit_bytes=64 * 1024 * 1024,
)


def _dot(a, b):
    return jnp.dot(a, b, preferred_element_type=_F32)


def _dot_ta(a, b):
    return jax.lax.dot_general(a, b, (((0,), (0,)), ((), ())),
                               preferred_element_type=_F32)


def _pass1_body(x_ref, w1_ref, b1_ref, s_ref, q_ref, *, bb):
    s = jnp.zeros((1, 128), _F32)
    q = jnp.zeros((1, 128), _F32)
    for i in range(bb):
        pre = _dot_ta(x_ref[i], w1_ref[...]) + b1_ref[...]
        s += jnp.sum(pre, axis=0, keepdims=True)
        q += jnp.sum(pre * pre, axis=0, keepdims=True)
    s_ref[0] = s
    q_ref[0] = q


def _pass2_body(x_ref, w1_ref, a1_ref, w2_ref, f_ref, o_ref, *, bb):
    sh = jnp.zeros((1, 128), _F32)
    q = jnp.zeros((1, 256), _F32)
    rows = []
    for i in range(bb):
        h1 = jnp.maximum(_dot_ta(x_ref[i], w1_ref[...]) + a1_ref[...], 0.0)
        sh += jnp.sum(h1, axis=0, keepdims=True)
        y = _dot(h1, w2_ref[...])
        f_ref[i] = y.astype(_BF16)
        q += jnp.sum(y * y, axis=0, keepdims=True)
        rows.append(jnp.max(y, axis=0, keepdims=True))
        rows.append(jnp.min(y, axis=0, keepdims=True))
    rows.append(q)
    rows.append(jnp.pad(sh, ((0, 0), (0, 128))))
    o_ref[0] = jnp.concatenate(rows, axis=0)


def _pass3_body(f_ref, w3_ref, gc_ref, q_ref, *, bb):
    q = jnp.zeros((1, 512), _F32)
    for i in range(bb):
        pre = _dot(f_ref[i], w3_ref[...]) + gc_ref[i]
        q += jnp.sum(pre * pre, axis=0, keepdims=True)
    q_ref[0] = q


def _pass4_body(f_ref, w3_ref, gc3_ref, w4_ref, o_ref, *, fd):
    h3 = jnp.maximum(_dot(f_ref[0], w3_ref[...]) + gc3_ref[0], 0.0)
    sh = jnp.sum(h3, axis=0, keepdims=True)
    y = _dot(h3.astype(_BF16), w4_ref[...])
    o_ref[0] = jnp.concatenate(
        [jnp.sum(y * y, axis=0, keepdims=True),
         jnp.max(y, axis=0, keepdims=True),
         jnp.min(y, axis=0, keepdims=True),
         jnp.pad(sh, ((0, 0), (0, fd - 512)))], axis=0)


def _grid2(g):
    nc = 2 if g % 2 == 0 else 1
    return (nc, g // nc), g // nc


def _row_spec(bb, n, c, h):
    return pl.BlockSpec((bb, n, c), lambda ci, j: (ci * h + j, 0, 0))


def _tile_spec(tn, c, nt, h):
    return pl.BlockSpec((1, tn, c),
                        lambda ci, j: ((ci * h + j) // nt, (ci * h + j) % nt, 0))


def _b_of_tile_spec(c, nt, h):
    return pl.BlockSpec((1, 1, c), lambda ci, j: ((ci * h + j) // nt, 0, 0))


def _per_b_spec(bb, c, h):
    return pl.BlockSpec((bb, 1, c), lambda ci, j: (ci * h + j, 0, 0))


def _step_spec(c, h):
    return pl.BlockSpec((1, 1, c), lambda ci, j: (ci * h + j, 0, 0))


def _full_spec(shape):
    return pl.BlockSpec(shape, lambda ci, j: (0,) * len(shape))


def _stat_shape(steps, c):
    return jax.ShapeDtypeStruct((steps, 1, c), _F32)


def _bn_fold(s, q, count, gamma, beta):
    mean = s / count
    var = jnp.maximum(q / count - mean * mean, 0.0)
    scale = gamma * jax.lax.rsqrt(var + _BN_EPS)
    return scale, beta - mean * scale


def _affine_max(scale, shift, mx, mn):
    return jnp.where(scale > 0, scale * mx + shift, scale * mn + shift)


@jax.jit
def _encode(x_ncw, p):
    B, c_in, N = x_ncw.shape
    fd = p["w4"].shape[1]
    count = jnp.float32(B * N)

    x = x_ncw
    w1 = p["w1"]
    b1, w2, b2, b3, w4, b4 = p["b1"], p["w2"], p["b2"], p["b3"], p["w4"], p["b4"]
    w3g, w3f = p["w3"][:256], p["w3"][256:]

    bb1 = min(16, B)
    g1 = B // bb1
    grid1, h1 = _grid2(g1)
    s1, q1 = pl.pallas_call(
        functools.partial(_pass1_body, bb=bb1),
        grid=grid1,
        in_specs=[_row_spec(bb1, c_in, N, h1), _full_spec((c_in, 128)),
                  _full_spec((1, 128))],
        out_specs=[_step_spec(128, h1), _step_spec(128, h1)],
        out_shape=(_stat_shape(g1, 128), _stat_shape(g1, 128)),
        compiler_params=_PARAMS,
    )(x, w1, b1)
    sc1, sf1 = _bn_fold(jnp.sum(s1, 0), jnp.sum(q1, 0), count,
                        p["g1"], p["be1"])
    w1f = w1 * sc1
    a1 = sc1 * b1 + sf1

    bb2 = min(8, B)
    g2 = B // bb2
    grid2, h2 = _grid2(g2)
    nrow2 = 2 * bb2 + 2
    feat, o2 = pl.pallas_call(
        functools.partial(_pass2_body, bb=bb2),
        grid=grid2,
        in_specs=[_row_spec(bb2, c_in, N, h2), _full_spec((c_in, 128)),
                  _full_spec((1, 128)), _full_spec((128, 256))],
        out_specs=[_row_spec(bb2, N, 256, h2),
                   pl.BlockSpec((1, nrow2, 256),
                                lambda ci, j, _h=h2: (ci * _h + j, 0, 0))],
        out_shape=(jax.ShapeDtypeStruct((B, N, 256), _BF16),
                   jax.ShapeDtypeStruct((g2, nrow2, 256), _F32)),
        compiler_params=_PARAMS,
    )(x, w1f, a1, w2)
    mxmn = o2[:, :2 * bb2, :].reshape(B, 2, 256)
    fmx = mxmn[:, 0, :] + b2
    fmn = mxmn[:, 1, :] + b2
    sh1 = jnp.sum(o2[:, 2 * bb2 + 1, :128], axis=0, keepdims=True)
    sy2 = jnp.dot(sh1, w2, precision=_HI)
    s2 = sy2 + count * b2
    q2 = jnp.sum(o2[:, 2 * bb2, :], 0) + 2.0 * b2 * sy2 + count * b2 * b2
    sc2, sf2 = _bn_fold(s2, q2, count, p["g2"], p["be2"])

    g = _affine_max(sc2, sf2, fmx, fmn)
    w3s = sc2.reshape(256, 1) * w3f
    gc = (jnp.dot(g, w3g, precision=_HI)
          + jnp.dot(sf2, w3f, precision=_HI) + b3
          + jnp.dot(b2, w3s, precision=_HI))
    gc = gc.reshape(B, 1, 512)

    bb3 = min(8, B)
    g3 = B // bb3
    grid3, h3 = _grid2(g3)
    (q3,) = pl.pallas_call(
        functools.partial(_pass3_body, bb=bb3),
        grid=grid3,
        in_specs=[_row_spec(bb3, N, 256, h3), _full_spec((256, 512)),
                  _per_b_spec(bb3, 512, h3)],
        out_specs=[_step_spec(512, h3)],
        out_shape=(_stat_shape(g3, 512),),
        compiler_params=_PARAMS,
    )(feat, w3s.astype(_BF16), gc)
    s3 = (jnp.dot(sy2.reshape(1, 256), w3s, precision=_HI)
          + N * jnp.sum(gc[:, 0, :], 0, keepdims=True))
    sc3, sf3 = _bn_fold(s3, jnp.sum(q3, 0), count, p["g3"], p["be3"])
    w34 = (w3s * sc3).astype(_BF16)
    gc3 = gc * sc3.reshape(1, 1, 512) + sf3.reshape(1, 1, 512)

    tn4 = min(2048, N)
    nt4 = N // tn4
    g4 = B * nt4
    grid4, h4 = _grid2(g4)
    (o4,) = pl.pallas_call(
        functools.partial(_pass4_body, fd=fd),
        grid=grid4,
        in_specs=[_tile_spec(tn4, 256, nt4, h4), _full_spec((256, 512)),
                  _b_of_tile_spec(512, nt4, h4), _full_spec((512, fd))],
        out_specs=[pl.BlockSpec((1, 4, fd),
                                lambda ci, j, _h=h4: (ci * _h + j, 0, 0))],
        out_shape=(jax.ShapeDtypeStruct((g4, 4, fd), _F32),),
        compiler_params=_PARAMS,
    )(feat, w34, gc3, w4.astype(_BF16))
    sh3 = jnp.sum(o4[:, 3, :512], axis=0, keepdims=True)
    sy = jnp.dot(sh3, w4, precision=_HI)
    s4 = sy + count * b4
    qy = jnp.sum(o4[:, 0, :], 0) + 2.0 * b4 * sy + count * b4 * b4
    sc4, sf4 = _bn_fold(s4, qy, count, p["g4"], p["be4"])

    hmx = jnp.max(o4[:, 1, :].reshape(B, nt4, fd), axis=1) + b4
    hmn = jnp.min(o4[:, 2, :].reshape(B, nt4, fd), axis=1) + b4
    return _affine_max(sc4, sf4, hmx, hmn)


def kernel(x, w1, b1, g1, be1, w2, b2, g2, be2,
           w3, b3, g3, be3, w4, b4, g4, be4):
    p = {
        "w1": w1, "b1": b1, "g1": g1, "be1": be1,
        "w2": w2, "b2": b2, "g2": g2, "be2": be2,
        "w3": w3, "b3": b3, "g3": g3, "be3": be3,
        "w4": w4, "b4": b4, "g4": g4, "be4": be4,
    }
    return _encode(x, p)


# speedup vs baseline: 1.1197x; 1.0434x over previous
import functools

import jax
import jax.numpy as jnp
from jax.experimental import pallas as pl
from jax.experimental.pallas import tpu as pltpu

_BN_EPS = 1e-5
_F32 = jnp.float32
_BF16 = jnp.bfloat16
_HI = jax.lax.Precision.HIGHEST

_PARAMS = pltpu.CompilerParams(
    dimension_semantics=("parallel", "arbitrary"),
    vmem_limit_bytes=64 * 1024 * 1024,
)


def _dot(a, b):
    return jnp.dot(a, b, preferred_element_type=_F32)


def _dot_ta(a, b):
    return jax.lax.dot_general(a, b, (((0,), (0,)), ((), ())),
                               preferred_element_type=_F32)


def _pass1_body(x_ref, w1_ref, b1_ref, s_ref, q_ref, *, bb):
    s = jnp.zeros((1, 128), _F32)
    q = jnp.zeros((1, 128), _F32)
    for i in range(bb):
        pre = _dot_ta(x_ref[i], w1_ref[...]) + b1_ref[...]
        s += jnp.sum(pre, axis=0, keepdims=True)
        q += jnp.sum(pre * pre, axis=0, keepdims=True)
    s_ref[0] = s
    q_ref[0] = q


def _pass2_body(x_ref, w1_ref, a1_ref, w2_ref, f_ref, o_ref, *, bb):
    sh = jnp.zeros((1, 128), _F32)
    q = jnp.zeros((1, 256), _F32)
    rows = []
    for i in range(bb):
        h1 = jnp.maximum(_dot_ta(x_ref[i], w1_ref[...]) + a1_ref[...], 0.0)
        sh += jnp.sum(h1, axis=0, keepdims=True)
        y = _dot(h1, w2_ref[...])
        f_ref[i] = y.astype(_BF16)
        q += jnp.sum(y * y, axis=0, keepdims=True)
        rows.append(jnp.max(y, axis=0, keepdims=True))
        rows.append(jnp.min(y, axis=0, keepdims=True))
    rows.append(q)
    rows.append(jnp.pad(sh, ((0, 0), (0, 128))))
    o_ref[0] = jnp.concatenate(rows, axis=0)


def _pass3_body(f_ref, w3_ref, gc_ref, q_ref, *, bb):
    q = jnp.zeros((1, 512), _F32)
    for i in range(bb):
        pre = _dot(f_ref[i], w3_ref[...]) + gc_ref[i]
        q += jnp.sum(pre * pre, axis=0, keepdims=True)
    q_ref[0] = q


def _pass4_body(f_ref, w3_ref, gc3_ref, w4_ref, o_ref, *, bb, fd):
    rows = []
    for i in range(bb):
        h3 = jnp.maximum(_dot(f_ref[i], w3_ref[...]) + gc3_ref[i], 0.0)
        sh = jnp.sum(h3, axis=0, keepdims=True)
        y = _dot(h3.astype(_BF16), w4_ref[...])
        rows.append(jnp.sum(y * y, axis=0, keepdims=True))
        rows.append(jnp.max(y, axis=0, keepdims=True))
        rows.append(jnp.min(y, axis=0, keepdims=True))
        rows.append(jnp.pad(sh, ((0, 0), (0, fd - 512))))
    o_ref[0] = jnp.concatenate(rows, axis=0)


def _grid2(g):
    nc = 2 if g % 2 == 0 else 1
    return (nc, g // nc), g // nc


def _row_spec(bb, n, c, h):
    return pl.BlockSpec((bb, n, c), lambda ci, j: (ci * h + j, 0, 0))


def _tile_spec(tn, c, nt, h):
    return pl.BlockSpec((1, tn, c),
                        lambda ci, j: ((ci * h + j) // nt, (ci * h + j) % nt, 0))


def _b_of_tile_spec(c, nt, h):
    return pl.BlockSpec((1, 1, c), lambda ci, j: ((ci * h + j) // nt, 0, 0))


def _per_b_spec(bb, c, h):
    return pl.BlockSpec((bb, 1, c), lambda ci, j: (ci * h + j, 0, 0))


def _step_spec(c, h):
    return pl.BlockSpec((1, 1, c), lambda ci, j: (ci * h + j, 0, 0))


def _full_spec(shape):
    return pl.BlockSpec(shape, lambda ci, j: (0,) * len(shape))


def _stat_shape(steps, c):
    return jax.ShapeDtypeStruct((steps, 1, c), _F32)


def _bn_fold(s, q, count, gamma, beta):
    mean = s / count
    var = jnp.maximum(q / count - mean * mean, 0.0)
    scale = gamma * jax.lax.rsqrt(var + _BN_EPS)
    return scale, beta - mean * scale


def _affine_max(scale, shift, mx, mn):
    return jnp.where(scale > 0, scale * mx + shift, scale * mn + shift)


@jax.jit
def _encode(x_ncw, p):
    B, c_in, N = x_ncw.shape
    fd = p["w4"].shape[1]
    count = jnp.float32(B * N)

    x = x_ncw
    w1 = p["w1"]
    b1, w2, b2, b3, w4, b4 = p["b1"], p["w2"], p["b2"], p["b3"], p["w4"], p["b4"]
    w3g, w3f = p["w3"][:256], p["w3"][256:]

    bb1 = min(16, B)
    g1 = B // bb1
    grid1, h1 = _grid2(g1)
    s1, q1 = pl.pallas_call(
        functools.partial(_pass1_body, bb=bb1),
        grid=grid1,
        in_specs=[_row_spec(bb1, c_in, N, h1), _full_spec((c_in, 128)),
                  _full_spec((1, 128))],
        out_specs=[_step_spec(128, h1), _step_spec(128, h1)],
        out_shape=(_stat_shape(g1, 128), _stat_shape(g1, 128)),
        compiler_params=_PARAMS,
    )(x, w1, b1)
    sc1, sf1 = _bn_fold(jnp.sum(s1, 0), jnp.sum(q1, 0), count,
                        p["g1"], p["be1"])
    w1f = w1 * sc1
    a1 = sc1 * b1 + sf1

    bb2 = min(8, B)
    g2 = B // bb2
    grid2, h2 = _grid2(g2)
    nrow2 = 2 * bb2 + 2
    feat, o2 = pl.pallas_call(
        functools.partial(_pass2_body, bb=bb2),
        grid=grid2,
        in_specs=[_row_spec(bb2, c_in, N, h2), _full_spec((c_in, 128)),
                  _full_spec((1, 128)), _full_spec((128, 256))],
        out_specs=[_row_spec(bb2, N, 256, h2),
                   pl.BlockSpec((1, nrow2, 256),
                                lambda ci, j, _h=h2: (ci * _h + j, 0, 0))],
        out_shape=(jax.ShapeDtypeStruct((B, N, 256), _BF16),
                   jax.ShapeDtypeStruct((g2, nrow2, 256), _F32)),
        compiler_params=_PARAMS,
    )(x, w1f, a1, w2)
    mxmn = o2[:, :2 * bb2, :].reshape(B, 2, 256)
    fmx = mxmn[:, 0, :] + b2
    fmn = mxmn[:, 1, :] + b2
    sh1 = jnp.sum(o2[:, 2 * bb2 + 1, :128], axis=0, keepdims=True)
    sy2 = jnp.dot(sh1, w2, precision=_HI)
    s2 = sy2 + count * b2
    q2 = jnp.sum(o2[:, 2 * bb2, :], 0) + 2.0 * b2 * sy2 + count * b2 * b2
    sc2, sf2 = _bn_fold(s2, q2, count, p["g2"], p["be2"])

    g = _affine_max(sc2, sf2, fmx, fmn)
    w3s = sc2.reshape(256, 1) * w3f
    gc = (jnp.dot(g, w3g, precision=_HI)
          + jnp.dot(sf2, w3f, precision=_HI) + b3
          + jnp.dot(b2, w3s, precision=_HI))
    gc = gc.reshape(B, 1, 512)

    bb3 = min(8, B)
    g3 = B // bb3
    grid3, h3 = _grid2(g3)
    (q3,) = pl.pallas_call(
        functools.partial(_pass3_body, bb=bb3),
        grid=grid3,
        in_specs=[_row_spec(bb3, N, 256, h3), _full_spec((256, 512)),
                  _per_b_spec(bb3, 512, h3)],
        out_specs=[_step_spec(512, h3)],
        out_shape=(_stat_shape(g3, 512),),
        compiler_params=_PARAMS,
    )(feat, w3s.astype(_BF16), gc)
    s3 = (jnp.dot(sy2.reshape(1, 256), w3s, precision=_HI)
          + N * jnp.sum(gc[:, 0, :], 0, keepdims=True))
    sc3, sf3 = _bn_fold(s3, jnp.sum(q3, 0), count, p["g3"], p["be3"])
    w34 = (w3s * sc3).astype(_BF16)
    gc3 = gc * sc3.reshape(1, 1, 512) + sf3.reshape(1, 1, 512)

    bb4 = min(2, B)
    nt4 = 1
    g4 = B // bb4
    grid4, h4 = _grid2(g4)
    (o4,) = pl.pallas_call(
        functools.partial(_pass4_body, bb=bb4, fd=fd),
        grid=grid4,
        in_specs=[_row_spec(bb4, N, 256, h4), _full_spec((256, 512)),
                  _per_b_spec(bb4, 512, h4), _full_spec((512, fd))],
        out_specs=[pl.BlockSpec((1, 4 * bb4, fd),
                                lambda ci, j, _h=h4: (ci * _h + j, 0, 0))],
        out_shape=(jax.ShapeDtypeStruct((g4, 4 * bb4, fd), _F32),),
        compiler_params=_PARAMS,
    )(feat, w34, gc3, w4.astype(_BF16))
    o4 = o4.reshape(B, 4, fd)
    sh3 = jnp.sum(o4[:, 3, :512], axis=0, keepdims=True)
    sy = jnp.dot(sh3, w4, precision=_HI)
    s4 = sy + count * b4
    qy = jnp.sum(o4[:, 0, :], 0) + 2.0 * b4 * sy + count * b4 * b4
    sc4, sf4 = _bn_fold(s4, qy, count, p["g4"], p["be4"])

    hmx = o4[:, 1, :] + b4
    hmn = o4[:, 2, :] + b4
    return _affine_max(sc4, sf4, hmx, hmn)


def kernel(x, w1, b1, g1, be1, w2, b2, g2, be2,
           w3, b3, g3, be3, w4, b4, g4, be4):
    p = {
        "w1": w1, "b1": b1, "g1": g1, "be1": be1,
        "w2": w2, "b2": b2, "g2": g2, "be2": be2,
        "w3": w3, "b3": b3, "g3": g3, "be3": be3,
        "w4": w4, "b4": b4, "g4": g4, "be4": be4,
    }
    return _encode(x, p)


# speedup vs baseline: 1.1401x; 1.0183x over previous
import functools

import jax
import jax.numpy as jnp
from jax.experimental import pallas as pl
from jax.experimental.pallas import tpu as pltpu

_BN_EPS = 1e-5
_F32 = jnp.float32
_BF16 = jnp.bfloat16
_HI = jax.lax.Precision.HIGHEST

_PARAMS = pltpu.CompilerParams(
    dimension_semantics=("parallel", "arbitrary"),
    vmem_limit_bytes=64 * 1024 * 1024,
)


def _dot(a, b):
    return jnp.dot(a, b, preferred_element_type=_F32)


def _dot_ta(a, b):
    return jax.lax.dot_general(a, b, (((0,), (0,)), ((), ())),
                               preferred_element_type=_F32)


def _pass1_body(x_ref, w1_ref, b1_ref, s_ref, q_ref, *, bb):
    s = jnp.zeros((1, 128), _F32)
    q = jnp.zeros((1, 128), _F32)
    for i in range(bb):
        pre = _dot_ta(x_ref[i], w1_ref[...]) + b1_ref[...]
        s += jnp.sum(pre, axis=0, keepdims=True)
        q += jnp.sum(pre * pre, axis=0, keepdims=True)
    s_ref[0] = s
    q_ref[0] = q


def _pass2_body(x_ref, w1_ref, a1_ref, w2_ref, f_ref, o_ref, *, bb):
    sh = jnp.zeros((1, 128), _F32)
    q = jnp.zeros((1, 256), _F32)
    rows = []
    for i in range(bb):
        h1 = jnp.maximum(_dot_ta(x_ref[i], w1_ref[...]) + a1_ref[...], 0.0)
        sh += jnp.sum(h1, axis=0, keepdims=True)
        y = _dot(h1, w2_ref[...])
        f_ref[i] = y.astype(_BF16)
        q += jnp.sum(y * y, axis=0, keepdims=True)
        rows.append(jnp.max(y, axis=0, keepdims=True))
        rows.append(jnp.min(y, axis=0, keepdims=True))
    rows.append(q)
    rows.append(jnp.pad(sh, ((0, 0), (0, 128))))
    o_ref[0] = jnp.concatenate(rows, axis=0)


def _pass3_body(f_ref, w3_ref, gc_ref, q_ref, *, bb):
    q = jnp.zeros((1, 512), _F32)
    for i in range(bb):
        pre = _dot(f_ref[i], w3_ref[...]) + gc_ref[i]
        q += jnp.sum(pre * pre, axis=0, keepdims=True)
    q_ref[0] = q


def _pass4_body(f_ref, w3_ref, gc3_ref, w4_ref, o_ref, *, bb, fd):
    rows = []
    for i in range(bb):
        h3 = jnp.maximum(_dot(f_ref[i], w3_ref[...]) + gc3_ref[i], 0.0)
        sh = jnp.sum(h3, axis=0, keepdims=True)
        y = _dot(h3.astype(_BF16), w4_ref[...])
        rows.append(jnp.sum(y * y, axis=0, keepdims=True))
        rows.append(jnp.max(y, axis=0, keepdims=True))
        rows.append(jnp.min(y, axis=0, keepdims=True))
        rows.append(jnp.pad(sh, ((0, 0), (0, fd - 512))))
    o_ref[0] = jnp.concatenate(rows, axis=0)


def _grid2(g):
    nc = 2 if g % 2 == 0 else 1
    return (nc, g // nc), g // nc


def _row_spec(bb, n, c, h):
    return pl.BlockSpec((bb, n, c), lambda ci, j: (ci * h + j, 0, 0))


def _tile_spec(tn, c, nt, h):
    return pl.BlockSpec((1, tn, c),
                        lambda ci, j: ((ci * h + j) // nt, (ci * h + j) % nt, 0))


def _b_of_tile_spec(c, nt, h):
    return pl.BlockSpec((1, 1, c), lambda ci, j: ((ci * h + j) // nt, 0, 0))


def _per_b_spec(bb, c, h):
    return pl.BlockSpec((bb, 1, c), lambda ci, j: (ci * h + j, 0, 0))


def _step_spec(c, h):
    return pl.BlockSpec((1, 1, c), lambda ci, j: (ci * h + j, 0, 0))


def _full_spec(shape):
    return pl.BlockSpec(shape, lambda ci, j: (0,) * len(shape))


def _stat_shape(steps, c):
    return jax.ShapeDtypeStruct((steps, 1, c), _F32)


def _bn_fold(s, q, count, gamma, beta):
    mean = s / count
    var = jnp.maximum(q / count - mean * mean, 0.0)
    scale = gamma * jax.lax.rsqrt(var + _BN_EPS)
    return scale, beta - mean * scale


def _affine_max(scale, shift, mx, mn):
    return jnp.where(scale > 0, scale * mx + shift, scale * mn + shift)


@jax.jit
def _encode(x_ncw, p):
    B, c_in, N = x_ncw.shape
    fd = p["w4"].shape[1]
    count = jnp.float32(B * N)

    x = x_ncw
    w1 = p["w1"]
    b1, w2, b2, b3, w4, b4 = p["b1"], p["w2"], p["b2"], p["b3"], p["w4"], p["b4"]
    w3g, w3f = p["w3"][:256], p["w3"][256:]

    bb1 = min(16, B)
    g1 = B // bb1
    grid1, h1 = _grid2(g1)
    s1, q1 = pl.pallas_call(
        functools.partial(_pass1_body, bb=bb1),
        grid=grid1,
        in_specs=[_row_spec(bb1, c_in, N, h1), _full_spec((c_in, 128)),
                  _full_spec((1, 128))],
        out_specs=[_step_spec(128, h1), _step_spec(128, h1)],
        out_shape=(_stat_shape(g1, 128), _stat_shape(g1, 128)),
        compiler_params=_PARAMS,
    )(x, w1, b1)
    sc1, sf1 = _bn_fold(jnp.sum(s1, 0), jnp.sum(q1, 0), count,
                        p["g1"], p["be1"])
    w1f = w1 * sc1
    a1 = sc1 * b1 + sf1

    bb2 = min(8, B)
    g2 = B // bb2
    grid2, h2 = _grid2(g2)
    nrow2 = 2 * bb2 + 2
    feat, o2 = pl.pallas_call(
        functools.partial(_pass2_body, bb=bb2),
        grid=grid2,
        in_specs=[_row_spec(bb2, c_in, N, h2), _full_spec((c_in, 128)),
                  _full_spec((1, 128)), _full_spec((128, 256))],
        out_specs=[_row_spec(bb2, N, 256, h2),
                   pl.BlockSpec((1, nrow2, 256),
                                lambda ci, j, _h=h2: (ci * _h + j, 0, 0))],
        out_shape=(jax.ShapeDtypeStruct((B, N, 256), _BF16),
                   jax.ShapeDtypeStruct((g2, nrow2, 256), _F32)),
        compiler_params=_PARAMS,
    )(x, w1f, a1, w2)
    mxmn = o2[:, :2 * bb2, :].reshape(B, 2, 256)
    fmx = mxmn[:, 0, :] + b2
    fmn = mxmn[:, 1, :] + b2
    sh1 = jnp.sum(o2[:, 2 * bb2 + 1, :128], axis=0, keepdims=True)
    sy2 = jnp.dot(sh1, w2, precision=_HI)
    s2 = sy2 + count * b2
    q2 = jnp.sum(o2[:, 2 * bb2, :], 0) + 2.0 * b2 * sy2 + count * b2 * b2
    sc2, sf2 = _bn_fold(s2, q2, count, p["g2"], p["be2"])

    g = _affine_max(sc2, sf2, fmx, fmn)
    w3s = sc2.reshape(256, 1) * w3f
    gc = (jnp.dot(g, w3g, precision=_HI)
          + jnp.dot(sf2, w3f, precision=_HI) + b3
          + jnp.dot(b2, w3s, precision=_HI))
    gc = gc.reshape(B, 1, 512)

    bb3 = min(8, B)
    g3 = B // bb3
    grid3, h3 = _grid2(g3)
    (q3,) = pl.pallas_call(
        functools.partial(_pass3_body, bb=bb3),
        grid=grid3,
        in_specs=[_row_spec(bb3, N, 256, h3), _full_spec((256, 512)),
                  _per_b_spec(bb3, 512, h3)],
        out_specs=[_step_spec(512, h3)],
        out_shape=(_stat_shape(g3, 512),),
        compiler_params=_PARAMS,
    )(feat, w3s.astype(_BF16), gc)
    s3 = (jnp.dot(sy2.reshape(1, 256), w3s, precision=_HI)
          + N * jnp.sum(gc[:, 0, :], 0, keepdims=True))
    sc3, sf3 = _bn_fold(s3, jnp.sum(q3, 0), count, p["g3"], p["be3"])
    w34 = (w3s * sc3).astype(_BF16)
    gc3 = gc * sc3.reshape(1, 1, 512) + sf3.reshape(1, 1, 512)

    bb4 = min(4, B)
    nt4 = 1
    g4 = B // bb4
    grid4, h4 = _grid2(g4)
    (o4,) = pl.pallas_call(
        functools.partial(_pass4_body, bb=bb4, fd=fd),
        grid=grid4,
        in_specs=[_row_spec(bb4, N, 256, h4), _full_spec((256, 512)),
                  _per_b_spec(bb4, 512, h4), _full_spec((512, fd))],
        out_specs=[pl.BlockSpec((1, 4 * bb4, fd),
                                lambda ci, j, _h=h4: (ci * _h + j, 0, 0))],
        out_shape=(jax.ShapeDtypeStruct((g4, 4 * bb4, fd), _F32),),
        compiler_params=_PARAMS,
    )(feat, w34, gc3, w4.astype(_BF16))
    o4 = o4.reshape(B, 4, fd)
    sh3 = jnp.sum(o4[:, 3, :512], axis=0, keepdims=True)
    sy = jnp.dot(sh3, w4, precision=_HI)
    s4 = sy + count * b4
    qy = jnp.sum(o4[:, 0, :], 0) + 2.0 * b4 * sy + count * b4 * b4
    sc4, sf4 = _bn_fold(s4, qy, count, p["g4"], p["be4"])

    hmx = o4[:, 1, :] + b4
    hmn = o4[:, 2, :] + b4
    return _affine_max(sc4, sf4, hmx, hmn)


def kernel(x, w1, b1, g1, be1, w2, b2, g2, be2,
           w3, b3, g3, be3, w4, b4, g4, be4):
    p = {
        "w1": w1, "b1": b1, "g1": g1, "be1": be1,
        "w2": w2, "b2": b2, "g2": g2, "be2": be2,
        "w3": w3, "b3": b3, "g3": g3, "be3": be3,
        "w4": w4, "b4": b4, "g4": g4, "be4": be4,
    }
    return _encode(x, p)


# speedup vs baseline: 1.1416x; 1.0013x over previous
import functools

import jax
import jax.numpy as jnp
from jax.experimental import pallas as pl
from jax.experimental.pallas import tpu as pltpu

_BN_EPS = 1e-5
_F32 = jnp.float32
_BF16 = jnp.bfloat16
_HI = jax.lax.Precision.HIGHEST

_PARAMS = pltpu.CompilerParams(
    dimension_semantics=("parallel", "arbitrary"),
    vmem_limit_bytes=64 * 1024 * 1024,
)


def _dot(a, b):
    return jnp.dot(a, b, preferred_element_type=_F32)


def _dot_ta(a, b):
    return jax.lax.dot_general(a, b, (((0,), (0,)), ((), ())),
                               preferred_element_type=_F32)


def _pass1_body(x_ref, w1_ref, b1_ref, s_ref, q_ref, *, bb):
    s = jnp.zeros((1, 128), _F32)
    q = jnp.zeros((1, 128), _F32)
    for i in range(bb):
        pre = _dot_ta(x_ref[i], w1_ref[...]) + b1_ref[...]
        s += jnp.sum(pre, axis=0, keepdims=True)
        q += jnp.sum(pre * pre, axis=0, keepdims=True)
    s_ref[0] = s
    q_ref[0] = q


def _pass2_body(x_ref, w1_ref, a1_ref, w2_ref, f_ref, o_ref, *, bb):
    sh = jnp.zeros((1, 128), _F32)
    q = jnp.zeros((1, 256), _F32)
    rows = []
    for i in range(bb):
        h1 = jnp.maximum(_dot_ta(x_ref[i], w1_ref[...]) + a1_ref[...], 0.0)
        sh += jnp.sum(h1, axis=0, keepdims=True)
        y = _dot(h1, w2_ref[...])
        f_ref[i] = y.astype(_BF16)
        q += jnp.sum(y * y, axis=0, keepdims=True)
        rows.append(jnp.max(y, axis=0, keepdims=True))
        rows.append(jnp.min(y, axis=0, keepdims=True))
    rows.append(q)
    rows.append(jnp.pad(sh, ((0, 0), (0, 128))))
    o_ref[0] = jnp.concatenate(rows, axis=0)


def _pass3_body(f_ref, w3_ref, gc_ref, q_ref, *, bb):
    q = jnp.zeros((1, 512), _F32)
    for i in range(bb):
        pre = _dot(f_ref[i], w3_ref[...]) + gc_ref[i]
        q += jnp.sum(pre * pre, axis=0, keepdims=True)
    q_ref[0] = q


def _pass4_body(f_ref, w3_ref, gc3_ref, w4_ref, o_ref, *, bb, fd):
    rows = []
    for i in range(bb):
        h3 = jnp.maximum(_dot(f_ref[i], w3_ref[...]) + gc3_ref[i], 0.0)
        sh = jnp.sum(h3, axis=0, keepdims=True)
        y = _dot(h3.astype(_BF16), w4_ref[...])
        rows.append(jnp.sum(y * y, axis=0, keepdims=True))
        rows.append(jnp.max(y, axis=0, keepdims=True))
        rows.append(jnp.min(y, axis=0, keepdims=True))
        rows.append(jnp.pad(sh, ((0, 0), (0, fd - 512))))
    o_ref[0] = jnp.concatenate(rows, axis=0)


def _grid2(g):
    nc = 2 if g % 2 == 0 else 1
    return (nc, g // nc), g // nc


def _row_spec(bb, n, c, h):
    return pl.BlockSpec((bb, n, c), lambda ci, j: (ci * h + j, 0, 0))


def _tile_spec(tn, c, nt, h):
    return pl.BlockSpec((1, tn, c),
                        lambda ci, j: ((ci * h + j) // nt, (ci * h + j) % nt, 0))


def _b_of_tile_spec(c, nt, h):
    return pl.BlockSpec((1, 1, c), lambda ci, j: ((ci * h + j) // nt, 0, 0))


def _per_b_spec(bb, c, h):
    return pl.BlockSpec((bb, 1, c), lambda ci, j: (ci * h + j, 0, 0))


def _step_spec(c, h):
    return pl.BlockSpec((1, 1, c), lambda ci, j: (ci * h + j, 0, 0))


def _full_spec(shape):
    return pl.BlockSpec(shape, lambda ci, j: (0,) * len(shape))


def _stat_shape(steps, c):
    return jax.ShapeDtypeStruct((steps, 1, c), _F32)


def _bn_fold(s, q, count, gamma, beta):
    mean = s / count
    var = jnp.maximum(q / count - mean * mean, 0.0)
    scale = gamma * jax.lax.rsqrt(var + _BN_EPS)
    return scale, beta - mean * scale


def _affine_max(scale, shift, mx, mn):
    return jnp.where(scale > 0, scale * mx + shift, scale * mn + shift)


@jax.jit
def _encode(x_ncw, p):
    B, c_in, N = x_ncw.shape
    fd = p["w4"].shape[1]
    count = jnp.float32(B * N)

    x = x_ncw
    w1 = p["w1"]
    b1, w2, b2, b3, w4, b4 = p["b1"], p["w2"], p["b2"], p["b3"], p["w4"], p["b4"]
    w3g, w3f = p["w3"][:256], p["w3"][256:]

    bb1 = min(16, B)
    g1 = B // bb1
    grid1, h1 = _grid2(g1)
    s1, q1 = pl.pallas_call(
        functools.partial(_pass1_body, bb=bb1),
        grid=grid1,
        in_specs=[_row_spec(bb1, c_in, N, h1), _full_spec((c_in, 128)),
                  _full_spec((1, 128))],
        out_specs=[_step_spec(128, h1), _step_spec(128, h1)],
        out_shape=(_stat_shape(g1, 128), _stat_shape(g1, 128)),
        compiler_params=_PARAMS,
    )(x, w1, b1)
    sc1, sf1 = _bn_fold(jnp.sum(s1, 0), jnp.sum(q1, 0), count,
                        p["g1"], p["be1"])
    w1f = w1 * sc1
    a1 = sc1 * b1 + sf1

    bb2 = min(8, B)
    g2 = B // bb2
    grid2, h2 = _grid2(g2)
    nrow2 = 2 * bb2 + 2
    feat, o2 = pl.pallas_call(
        functools.partial(_pass2_body, bb=bb2),
        grid=grid2,
        in_specs=[_row_spec(bb2, c_in, N, h2), _full_spec((c_in, 128)),
                  _full_spec((1, 128)), _full_spec((128, 256))],
        out_specs=[_row_spec(bb2, N, 256, h2),
                   pl.BlockSpec((1, nrow2, 256),
                                lambda ci, j, _h=h2: (ci * _h + j, 0, 0))],
        out_shape=(jax.ShapeDtypeStruct((B, N, 256), _BF16),
                   jax.ShapeDtypeStruct((g2, nrow2, 256), _F32)),
        compiler_params=_PARAMS,
    )(x, w1f, a1, w2)
    mxmn = o2[:, :2 * bb2, :].reshape(B, 2, 256)
    fmx = mxmn[:, 0, :] + b2
    fmn = mxmn[:, 1, :] + b2
    sh1 = jnp.sum(o2[:, 2 * bb2 + 1, :128], axis=0, keepdims=True)
    sy2 = jnp.dot(sh1, w2, precision=_HI)
    s2 = sy2 + count * b2
    q2 = jnp.sum(o2[:, 2 * bb2, :], 0) + 2.0 * b2 * sy2 + count * b2 * b2
    sc2, sf2 = _bn_fold(s2, q2, count, p["g2"], p["be2"])

    g = _affine_max(sc2, sf2, fmx, fmn)
    w3s = sc2.reshape(256, 1) * w3f
    gc = (jnp.dot(g, w3g, precision=_HI)
          + jnp.dot(sf2, w3f, precision=_HI) + b3
          + jnp.dot(b2, w3s, precision=_HI))
    gc = gc.reshape(B, 1, 512)

    bb3 = min(8, B)
    g3 = B // bb3
    grid3, h3 = _grid2(g3)
    (q3,) = pl.pallas_call(
        functools.partial(_pass3_body, bb=bb3),
        grid=grid3,
        in_specs=[_row_spec(bb3, N, 256, h3), _full_spec((256, 512)),
                  _per_b_spec(bb3, 512, h3)],
        out_specs=[_step_spec(512, h3)],
        out_shape=(_stat_shape(g3, 512),),
        compiler_params=_PARAMS,
    )(feat, w3s.astype(_BF16), gc)
    s3 = (jnp.dot(sy2.reshape(1, 256), w3s, precision=_HI)
          + N * jnp.sum(gc[:, 0, :], 0, keepdims=True))
    sc3, sf3 = _bn_fold(s3, jnp.sum(q3, 0), count, p["g3"], p["be3"])
    w34 = (w3s * sc3).astype(_BF16)
    gc3 = gc * sc3.reshape(1, 1, 512) + sf3.reshape(1, 1, 512)

    bb4 = min(8, B)
    nt4 = 1
    g4 = B // bb4
    grid4, h4 = _grid2(g4)
    (o4,) = pl.pallas_call(
        functools.partial(_pass4_body, bb=bb4, fd=fd),
        grid=grid4,
        in_specs=[_row_spec(bb4, N, 256, h4), _full_spec((256, 512)),
                  _per_b_spec(bb4, 512, h4), _full_spec((512, fd))],
        out_specs=[pl.BlockSpec((1, 4 * bb4, fd),
                                lambda ci, j, _h=h4: (ci * _h + j, 0, 0))],
        out_shape=(jax.ShapeDtypeStruct((g4, 4 * bb4, fd), _F32),),
        compiler_params=_PARAMS,
    )(feat, w34, gc3, w4.astype(_BF16))
    o4 = o4.reshape(B, 4, fd)
    sh3 = jnp.sum(o4[:, 3, :512], axis=0, keepdims=True)
    sy = jnp.dot(sh3, w4, precision=_HI)
    s4 = sy + count * b4
    qy = jnp.sum(o4[:, 0, :], 0) + 2.0 * b4 * sy + count * b4 * b4
    sc4, sf4 = _bn_fold(s4, qy, count, p["g4"], p["be4"])

    hmx = o4[:, 1, :] + b4
    hmn = o4[:, 2, :] + b4
    return _affine_max(sc4, sf4, hmx, hmn)


def kernel(x, w1, b1, g1, be1, w2, b2, g2, be2,
           w3, b3, g3, be3, w4, b4, g4, be4):
    p = {
        "w1": w1, "b1": b1, "g1": g1, "be1": be1,
        "w2": w2, "b2": b2, "g2": g2, "be2": be2,
        "w3": w3, "b3": b3, "g3": g3, "be3": be3,
        "w4": w4, "b4": b4, "g4": g4, "be4": be4,
    }
    return _encode(x, p)


# speedup vs baseline: 1.1517x; 1.0088x over previous
import functools

import jax
import jax.numpy as jnp
from jax.experimental import pallas as pl
from jax.experimental.pallas import tpu as pltpu

_BN_EPS = 1e-5
_F32 = jnp.float32
_BF16 = jnp.bfloat16
_HI = jax.lax.Precision.HIGHEST

_PARAMS = pltpu.CompilerParams(
    dimension_semantics=("parallel", "arbitrary"),
    vmem_limit_bytes=64 * 1024 * 1024,
)


def _dot(a, b):
    return jnp.dot(a, b, preferred_element_type=_F32)


def _dot_ta(a, b):
    return jax.lax.dot_general(a, b, (((0,), (0,)), ((), ())),
                               preferred_element_type=_F32)


def _pass1_body(x_ref, w1_ref, b1_ref, s_ref, q_ref, *, bb):
    s = jnp.zeros((1, 128), _F32)
    q = jnp.zeros((1, 128), _F32)
    for i in range(bb):
        pre = _dot_ta(x_ref[i], w1_ref[...]) + b1_ref[...]
        s += jnp.sum(pre, axis=0, keepdims=True)
        q += jnp.sum(pre * pre, axis=0, keepdims=True)
    s_ref[0] = s
    q_ref[0] = q


def _pass2_body(x_ref, w1_ref, a1_ref, w2_ref, f_ref, o_ref, *, bb):
    sh = jnp.zeros((1, 128), _F32)
    q = jnp.zeros((1, 256), _F32)
    rows = []
    for i in range(bb):
        h1 = jnp.maximum(_dot_ta(x_ref[i], w1_ref[...]) + a1_ref[...], 0.0)
        sh += jnp.sum(h1, axis=0, keepdims=True)
        y = _dot(h1, w2_ref[...])
        f_ref[i] = y.astype(_BF16)
        q += jnp.sum(y * y, axis=0, keepdims=True)
        rows.append(jnp.max(y, axis=0, keepdims=True))
        rows.append(jnp.min(y, axis=0, keepdims=True))
    rows.append(q)
    rows.append(jnp.pad(sh, ((0, 0), (0, 128))))
    o_ref[0] = jnp.concatenate(rows, axis=0)


def _pass3_body(f_ref, w3_ref, gc_ref, q_ref, *, bb):
    q = jnp.zeros((1, 512), _F32)
    for i in range(bb):
        pre = _dot(f_ref[i], w3_ref[...]) + gc_ref[i]
        q += jnp.sum(pre * pre, axis=0, keepdims=True)
    q_ref[0] = q


def _pass4_body(f_ref, w3_ref, gc3_ref, w4_ref, o_ref, *, bb, fd):
    rows = []
    for i in range(bb):
        h3 = jnp.maximum(_dot(f_ref[i], w3_ref[...]) + gc3_ref[i], 0.0)
        sh = jnp.sum(h3, axis=0, keepdims=True)
        y = _dot(h3.astype(_BF16), w4_ref[...])
        rows.append(jnp.sum(y * y, axis=0, keepdims=True))
        rows.append(jnp.max(y, axis=0, keepdims=True))
        rows.append(jnp.min(y, axis=0, keepdims=True))
        rows.append(jnp.pad(sh, ((0, 0), (0, fd - 512))))
    o_ref[0] = jnp.concatenate(rows, axis=0)


def _grid2(g):
    nc = 2 if g % 2 == 0 else 1
    return (nc, g // nc), g // nc


def _row_spec(bb, n, c, h):
    return pl.BlockSpec((bb, n, c), lambda ci, j: (ci * h + j, 0, 0))


def _tile_spec(tn, c, nt, h):
    return pl.BlockSpec((1, tn, c),
                        lambda ci, j: ((ci * h + j) // nt, (ci * h + j) % nt, 0))


def _b_of_tile_spec(c, nt, h):
    return pl.BlockSpec((1, 1, c), lambda ci, j: ((ci * h + j) // nt, 0, 0))


def _per_b_spec(bb, c, h):
    return pl.BlockSpec((bb, 1, c), lambda ci, j: (ci * h + j, 0, 0))


def _step_spec(c, h):
    return pl.BlockSpec((1, 1, c), lambda ci, j: (ci * h + j, 0, 0))


def _full_spec(shape):
    return pl.BlockSpec(shape, lambda ci, j: (0,) * len(shape))


def _stat_shape(steps, c):
    return jax.ShapeDtypeStruct((steps, 1, c), _F32)


def _bn_fold(s, q, count, gamma, beta):
    mean = s / count
    var = jnp.maximum(q / count - mean * mean, 0.0)
    scale = gamma * jax.lax.rsqrt(var + _BN_EPS)
    return scale, beta - mean * scale


def _affine_max(scale, shift, mx, mn):
    return jnp.where(scale > 0, scale * mx + shift, scale * mn + shift)


@jax.jit
def _encode(x_ncw, p):
    B, c_in, N = x_ncw.shape
    fd = p["w4"].shape[1]
    count = jnp.float32(B * N)

    x = x_ncw
    w1 = p["w1"]
    b1, w2, b2, b3, w4, b4 = p["b1"], p["w2"], p["b2"], p["b3"], p["w4"], p["b4"]
    w3g, w3f = p["w3"][:256], p["w3"][256:]

    bb1 = min(16, B)
    g1 = B // bb1
    grid1, h1 = _grid2(g1)
    s1, q1 = pl.pallas_call(
        functools.partial(_pass1_body, bb=bb1),
        grid=grid1,
        in_specs=[_row_spec(bb1, c_in, N, h1), _full_spec((c_in, 128)),
                  _full_spec((1, 128))],
        out_specs=[_step_spec(128, h1), _step_spec(128, h1)],
        out_shape=(_stat_shape(g1, 128), _stat_shape(g1, 128)),
        compiler_params=_PARAMS,
    )(x, w1, b1)
    sc1, sf1 = _bn_fold(jnp.sum(s1, 0), jnp.sum(q1, 0), count,
                        p["g1"], p["be1"])
    w1f = w1 * sc1
    a1 = sc1 * b1 + sf1

    bb2 = min(16, B)
    g2 = B // bb2
    grid2, h2 = _grid2(g2)
    nrow2 = 2 * bb2 + 2
    feat, o2 = pl.pallas_call(
        functools.partial(_pass2_body, bb=bb2),
        grid=grid2,
        in_specs=[_row_spec(bb2, c_in, N, h2), _full_spec((c_in, 128)),
                  _full_spec((1, 128)), _full_spec((128, 256))],
        out_specs=[_row_spec(bb2, N, 256, h2),
                   pl.BlockSpec((1, nrow2, 256),
                                lambda ci, j, _h=h2: (ci * _h + j, 0, 0))],
        out_shape=(jax.ShapeDtypeStruct((B, N, 256), _BF16),
                   jax.ShapeDtypeStruct((g2, nrow2, 256), _F32)),
        compiler_params=_PARAMS,
    )(x, w1f, a1, w2)
    mxmn = o2[:, :2 * bb2, :].reshape(B, 2, 256)
    fmx = mxmn[:, 0, :] + b2
    fmn = mxmn[:, 1, :] + b2
    sh1 = jnp.sum(o2[:, 2 * bb2 + 1, :128], axis=0, keepdims=True)
    sy2 = jnp.dot(sh1, w2, precision=_HI)
    s2 = sy2 + count * b2
    q2 = jnp.sum(o2[:, 2 * bb2, :], 0) + 2.0 * b2 * sy2 + count * b2 * b2
    sc2, sf2 = _bn_fold(s2, q2, count, p["g2"], p["be2"])

    g = _affine_max(sc2, sf2, fmx, fmn)
    w3s = sc2.reshape(256, 1) * w3f
    gc = (jnp.dot(g, w3g, precision=_HI)
          + jnp.dot(sf2, w3f, precision=_HI) + b3
          + jnp.dot(b2, w3s, precision=_HI))
    gc = gc.reshape(B, 1, 512)

    bb3 = min(8, B)
    g3 = B // bb3
    grid3, h3 = _grid2(g3)
    (q3,) = pl.pallas_call(
        functools.partial(_pass3_body, bb=bb3),
        grid=grid3,
        in_specs=[_row_spec(bb3, N, 256, h3), _full_spec((256, 512)),
                  _per_b_spec(bb3, 512, h3)],
        out_specs=[_step_spec(512, h3)],
        out_shape=(_stat_shape(g3, 512),),
        compiler_params=_PARAMS,
    )(feat, w3s.astype(_BF16), gc)
    s3 = (jnp.dot(sy2.reshape(1, 256), w3s, precision=_HI)
          + N * jnp.sum(gc[:, 0, :], 0, keepdims=True))
    sc3, sf3 = _bn_fold(s3, jnp.sum(q3, 0), count, p["g3"], p["be3"])
    w34 = (w3s * sc3).astype(_BF16)
    gc3 = gc * sc3.reshape(1, 1, 512) + sf3.reshape(1, 1, 512)

    bb4 = min(8, B)
    nt4 = 1
    g4 = B // bb4
    grid4, h4 = _grid2(g4)
    (o4,) = pl.pallas_call(
        functools.partial(_pass4_body, bb=bb4, fd=fd),
        grid=grid4,
        in_specs=[_row_spec(bb4, N, 256, h4), _full_spec((256, 512)),
                  _per_b_spec(bb4, 512, h4), _full_spec((512, fd))],
        out_specs=[pl.BlockSpec((1, 4 * bb4, fd),
                                lambda ci, j, _h=h4: (ci * _h + j, 0, 0))],
        out_shape=(jax.ShapeDtypeStruct((g4, 4 * bb4, fd), _F32),),
        compiler_params=_PARAMS,
    )(feat, w34, gc3, w4.astype(_BF16))
    o4 = o4.reshape(B, 4, fd)
    sh3 = jnp.sum(o4[:, 3, :512], axis=0, keepdims=True)
    sy = jnp.dot(sh3, w4, precision=_HI)
    s4 = sy + count * b4
    qy = jnp.sum(o4[:, 0, :], 0) + 2.0 * b4 * sy + count * b4 * b4
    sc4, sf4 = _bn_fold(s4, qy, count, p["g4"], p["be4"])

    hmx = o4[:, 1, :] + b4
    hmn = o4[:, 2, :] + b4
    return _affine_max(sc4, sf4, hmx, hmn)


def kernel(x, w1, b1, g1, be1, w2, b2, g2, be2,
           w3, b3, g3, be3, w4, b4, g4, be4):
    p = {
        "w1": w1, "b1": b1, "g1": g1, "be1": be1,
        "w2": w2, "b2": b2, "g2": g2, "be2": be2,
        "w3": w3, "b3": b3, "g3": g3, "be3": be3,
        "w4": w4, "b4": b4, "g4": g4, "be4": be4,
    }
    return _encode(x, p)


# speedup vs baseline: 1.1537x; 1.0017x over previous
import functools

import jax
import jax.numpy as jnp
from jax.experimental import pallas as pl
from jax.experimental.pallas import tpu as pltpu

_BN_EPS = 1e-5
_F32 = jnp.float32
_BF16 = jnp.bfloat16
_HI = jax.lax.Precision.HIGHEST

_PARAMS = pltpu.CompilerParams(
    dimension_semantics=("parallel", "arbitrary"),
    vmem_limit_bytes=64 * 1024 * 1024,
)


def _dot(a, b):
    return jnp.dot(a, b, preferred_element_type=_F32)


def _dot_ta(a, b):
    return jax.lax.dot_general(a, b, (((0,), (0,)), ((), ())),
                               preferred_element_type=_F32)


def _pass1_body(x_ref, w1_ref, b1_ref, s_ref, q_ref, *, bb):
    s = jnp.zeros((1, 128), _F32)
    q = jnp.zeros((1, 128), _F32)
    for i in range(bb):
        pre = _dot_ta(x_ref[i], w1_ref[...]) + b1_ref[...]
        s += jnp.sum(pre, axis=0, keepdims=True)
        q += jnp.sum(pre * pre, axis=0, keepdims=True)
    s_ref[0] = s
    q_ref[0] = q


def _pass2_body(x_ref, w1_ref, a1_ref, w2_ref, f_ref, o_ref, *, bb):
    sh = jnp.zeros((1, 128), _F32)
    q = jnp.zeros((1, 256), _F32)
    rows = []
    for i in range(bb):
        h1 = jnp.maximum(_dot_ta(x_ref[i], w1_ref[...]) + a1_ref[...], 0.0)
        sh += jnp.sum(h1, axis=0, keepdims=True)
        y = _dot(h1, w2_ref[...])
        f_ref[i] = y.astype(_BF16)
        q += jnp.sum(y * y, axis=0, keepdims=True)
        rows.append(jnp.max(y, axis=0, keepdims=True))
        rows.append(jnp.min(y, axis=0, keepdims=True))
    rows.append(q)
    rows.append(jnp.pad(sh, ((0, 0), (0, 128))))
    o_ref[0] = jnp.concatenate(rows, axis=0)


def _pass3_body(f_ref, w3_ref, gc_ref, q_ref, *, bb):
    q = jnp.zeros((1, 512), _F32)
    for i in range(bb):
        pre = _dot(f_ref[i], w3_ref[...]) + gc_ref[i]
        q += jnp.sum(pre * pre, axis=0, keepdims=True)
    q_ref[0] = q


def _pass4_body(f_ref, w3_ref, gc3_ref, w4_ref, o_ref, *, bb, fd):
    rows = []
    for i in range(bb):
        h3 = jnp.maximum(_dot(f_ref[i], w3_ref[...]) + gc3_ref[i], 0.0)
        sh = jnp.sum(h3, axis=0, keepdims=True)
        y = _dot(h3.astype(_BF16), w4_ref[...])
        rows.append(jnp.sum(y * y, axis=0, keepdims=True))
        rows.append(jnp.max(y, axis=0, keepdims=True))
        rows.append(jnp.min(y, axis=0, keepdims=True))
        rows.append(jnp.pad(sh, ((0, 0), (0, fd - 512))))
    o_ref[0] = jnp.concatenate(rows, axis=0)


def _grid2(g):
    nc = 2 if g % 2 == 0 else 1
    return (nc, g // nc), g // nc


def _row_spec(bb, n, c, h):
    return pl.BlockSpec((bb, n, c), lambda ci, j: (ci * h + j, 0, 0))


def _tile_spec(tn, c, nt, h):
    return pl.BlockSpec((1, tn, c),
                        lambda ci, j: ((ci * h + j) // nt, (ci * h + j) % nt, 0))


def _b_of_tile_spec(c, nt, h):
    return pl.BlockSpec((1, 1, c), lambda ci, j: ((ci * h + j) // nt, 0, 0))


def _per_b_spec(bb, c, h):
    return pl.BlockSpec((bb, 1, c), lambda ci, j: (ci * h + j, 0, 0))


def _step_spec(c, h):
    return pl.BlockSpec((1, 1, c), lambda ci, j: (ci * h + j, 0, 0))


def _full_spec(shape):
    return pl.BlockSpec(shape, lambda ci, j: (0,) * len(shape))


def _stat_shape(steps, c):
    return jax.ShapeDtypeStruct((steps, 1, c), _F32)


def _bn_fold(s, q, count, gamma, beta):
    mean = s / count
    var = jnp.maximum(q / count - mean * mean, 0.0)
    scale = gamma * jax.lax.rsqrt(var + _BN_EPS)
    return scale, beta - mean * scale


def _affine_max(scale, shift, mx, mn):
    return jnp.where(scale > 0, scale * mx + shift, scale * mn + shift)


@jax.jit
def _encode(x_ncw, p):
    B, c_in, N = x_ncw.shape
    fd = p["w4"].shape[1]
    count = jnp.float32(B * N)

    x = x_ncw
    w1 = p["w1"]
    b1, w2, b2, b3, w4, b4 = p["b1"], p["w2"], p["b2"], p["b3"], p["w4"], p["b4"]
    w3g, w3f = p["w3"][:256], p["w3"][256:]

    bb1 = min(40, B)
    g1 = B // bb1
    grid1, h1 = _grid2(g1)
    s1, q1 = pl.pallas_call(
        functools.partial(_pass1_body, bb=bb1),
        grid=grid1,
        in_specs=[_row_spec(bb1, c_in, N, h1), _full_spec((c_in, 128)),
                  _full_spec((1, 128))],
        out_specs=[_step_spec(128, h1), _step_spec(128, h1)],
        out_shape=(_stat_shape(g1, 128), _stat_shape(g1, 128)),
        compiler_params=_PARAMS,
    )(x, w1, b1)
    sc1, sf1 = _bn_fold(jnp.sum(s1, 0), jnp.sum(q1, 0), count,
                        p["g1"], p["be1"])
    w1f = w1 * sc1
    a1 = sc1 * b1 + sf1

    bb2 = min(16, B)
    g2 = B // bb2
    grid2, h2 = _grid2(g2)
    nrow2 = 2 * bb2 + 2
    feat, o2 = pl.pallas_call(
        functools.partial(_pass2_body, bb=bb2),
        grid=grid2,
        in_specs=[_row_spec(bb2, c_in, N, h2), _full_spec((c_in, 128)),
                  _full_spec((1, 128)), _full_spec((128, 256))],
        out_specs=[_row_spec(bb2, N, 256, h2),
                   pl.BlockSpec((1, nrow2, 256),
                                lambda ci, j, _h=h2: (ci * _h + j, 0, 0))],
        out_shape=(jax.ShapeDtypeStruct((B, N, 256), _BF16),
                   jax.ShapeDtypeStruct((g2, nrow2, 256), _F32)),
        compiler_params=_PARAMS,
    )(x, w1f, a1, w2)
    mxmn = o2[:, :2 * bb2, :].reshape(B, 2, 256)
    fmx = mxmn[:, 0, :] + b2
    fmn = mxmn[:, 1, :] + b2
    sh1 = jnp.sum(o2[:, 2 * bb2 + 1, :128], axis=0, keepdims=True)
    sy2 = jnp.dot(sh1, w2, precision=_HI)
    s2 = sy2 + count * b2
    q2 = jnp.sum(o2[:, 2 * bb2, :], 0) + 2.0 * b2 * sy2 + count * b2 * b2
    sc2, sf2 = _bn_fold(s2, q2, count, p["g2"], p["be2"])

    g = _affine_max(sc2, sf2, fmx, fmn)
    w3s = sc2.reshape(256, 1) * w3f
    gc = (jnp.dot(g, w3g, precision=_HI)
          + jnp.dot(sf2, w3f, precision=_HI) + b3
          + jnp.dot(b2, w3s, precision=_HI))
    gc = gc.reshape(B, 1, 512)

    bb3 = min(8, B)
    g3 = B // bb3
    grid3, h3 = _grid2(g3)
    (q3,) = pl.pallas_call(
        functools.partial(_pass3_body, bb=bb3),
        grid=grid3,
        in_specs=[_row_spec(bb3, N, 256, h3), _full_spec((256, 512)),
                  _per_b_spec(bb3, 512, h3)],
        out_specs=[_step_spec(512, h3)],
        out_shape=(_stat_shape(g3, 512),),
        compiler_params=_PARAMS,
    )(feat, w3s.astype(_BF16), gc)
    s3 = (jnp.dot(sy2.reshape(1, 256), w3s, precision=_HI)
          + N * jnp.sum(gc[:, 0, :], 0, keepdims=True))
    sc3, sf3 = _bn_fold(s3, jnp.sum(q3, 0), count, p["g3"], p["be3"])
    w34 = (w3s * sc3).astype(_BF16)
    gc3 = gc * sc3.reshape(1, 1, 512) + sf3.reshape(1, 1, 512)

    bb4 = min(8, B)
    nt4 = 1
    g4 = B // bb4
    grid4, h4 = _grid2(g4)
    (o4,) = pl.pallas_call(
        functools.partial(_pass4_body, bb=bb4, fd=fd),
        grid=grid4,
        in_specs=[_row_spec(bb4, N, 256, h4), _full_spec((256, 512)),
                  _per_b_spec(bb4, 512, h4), _full_spec((512, fd))],
        out_specs=[pl.BlockSpec((1, 4 * bb4, fd),
                                lambda ci, j, _h=h4: (ci * _h + j, 0, 0))],
        out_shape=(jax.ShapeDtypeStruct((g4, 4 * bb4, fd), _F32),),
        compiler_params=_PARAMS,
    )(feat, w34, gc3, w4.astype(_BF16))
    o4 = o4.reshape(B, 4, fd)
    sh3 = jnp.sum(o4[:, 3, :512], axis=0, keepdims=True)
    sy = jnp.dot(sh3, w4, precision=_HI)
    s4 = sy + count * b4
    qy = jnp.sum(o4[:, 0, :], 0) + 2.0 * b4 * sy + count * b4 * b4
    sc4, sf4 = _bn_fold(s4, qy, count, p["g4"], p["be4"])

    hmx = o4[:, 1, :] + b4
    hmn = o4[:, 2, :] + b4
    return _affine_max(sc4, sf4, hmx, hmn)


def kernel(x, w1, b1, g1, be1, w2, b2, g2, be2,
           w3, b3, g3, be3, w4, b4, g4, be4):
    p = {
        "w1": w1, "b1": b1, "g1": g1, "be1": be1,
        "w2": w2, "b2": b2, "g2": g2, "be2": be2,
        "w3": w3, "b3": b3, "g3": g3, "be3": be3,
        "w4": w4, "b4": b4, "g4": g4, "be4": be4,
    }
    return _encode(x, p)
